# baseline (speedup 1.0000x reference)
import functools

import jax
import jax.numpy as jnp
from jax import lax
from jax.experimental import pallas as pl
from jax.experimental.pallas import tpu as pltpu
from jax.experimental.pallas import tpu_sc as plsc

E = 8
TOPK = 2
BT = 128
I_BLK = 1024

NC = 2
NS = 16
NW = NC * NS


def _logits_body(x_ref, wg_ref, bg_ref, o_ref):
    o_ref[...] = (
        jax.lax.dot_general(
            x_ref[...], wg_ref[...],
            dimension_numbers=(((1,), (1,)), ((), ())),
            preferred_element_type=jnp.float32,
        )
        + bg_ref[...]
    )


def _gating_logits(x, Wg, bg):
    N, H = x.shape
    TB = 256
    return pl.pallas_call(
        _logits_body,
        grid=(N // TB,),
        in_specs=[
            pl.BlockSpec((TB, H), lambda b: (b, 0)),
            pl.BlockSpec((E, H), lambda b: (0, 0)),
            pl.BlockSpec((1, E), lambda b: (0, 0)),
        ],
        out_specs=pl.BlockSpec((TB, E), lambda b: (b, 0)),
        out_shape=jax.ShapeDtypeStruct((N, E), jnp.float32),
    )(x, Wg, bg.reshape(1, E))


def _sc_gather(x, idx, R, H):
    rows_per_w = R // NW
    CH = 48 if rows_per_w % 48 == 0 else 32
    nchunk = rows_per_w // CH
    mesh = plsc.VectorSubcoreMesh(core_axis_name="c", subcore_axis_name="s")

    @functools.partial(
        pl.kernel,
        mesh=mesh,
        out_type=jax.ShapeDtypeStruct((R, H), jnp.float32),
        scratch_types=[
            pltpu.VMEM((CH,), jnp.int32),
            pltpu.VMEM((CH, H), jnp.float32),
            pltpu.SemaphoreType.DMA,
        ],
    )
    def gk(x_hbm, idx_hbm, out_hbm, idx_v, rows_v, sem):
        wid = lax.axis_index("s") * NC + lax.axis_index("c")
        base = wid * rows_per_w

        def chunk(c, carry):
            off = base + c * CH
            pltpu.sync_copy(idx_hbm.at[pl.ds(off, CH)], idx_v)
            pltpu.async_copy(x_hbm.at[idx_v], rows_v, sem).wait()
            pltpu.sync_copy(rows_v, out_hbm.at[pl.ds(off, CH)])
            return carry

        lax.fori_loop(0, nchunk, chunk, 0)

    return gk(x, idx)


def _grouped_body(be_ref, xs_ref, wg_ref, wu_ref, wd_ref, g_ref,
                  yout_ref, *, ni):
    i = pl.program_id(1)
    xb = xs_ref[...]
    g1 = jax.lax.dot_general(
        xb, wg_ref[0], (((1,), (1,)), ((), ())),
        preferred_element_type=jnp.float32)
    u = jax.lax.dot_general(
        xb, wu_ref[0], (((1,), (1,)), ((), ())),
        preferred_element_type=jnp.float32)
    h = (g1 * jax.nn.sigmoid(g1)) * u
    part = jax.lax.dot_general(
        h, wd_ref[0], (((1,), (1,)), ((), ())),
        preferred_element_type=jnp.float32)

    @pl.when(i == 0)
    def _init():
        yout_ref[...] = part

    @pl.when(i != 0)
    def _acc():
        yout_ref[...] += part

    @pl.when(i == ni - 1)
    def _scale():
        yout_ref[...] *= g_ref[...]


def _grouped_mlp(block_expert, Xs, Wgate, Wup, Wdown, gate_sorted, R, H, I):
    NB = R // BT
    ni = I // I_BLK
    grid_spec = pltpu.PrefetchScalarGridSpec(
        num_scalar_prefetch=1,
        grid=(NB, ni),
        in_specs=[
            pl.BlockSpec((BT, H), lambda b, i, be: (b, 0)),
            pl.BlockSpec((1, I_BLK, H), lambda b, i, be: (be[b], i, 0)),
            pl.BlockSpec((1, I_BLK, H), lambda b, i, be: (be[b], i, 0)),
            pl.BlockSpec((1, H, I_BLK), lambda b, i, be: (be[b], 0, i)),
            pl.BlockSpec((BT, 1), lambda b, i, be: (b, 0)),
        ],
        out_specs=pl.BlockSpec((BT, H), lambda b, i, be: (b, 0)),
    )
    return pl.pallas_call(
        functools.partial(_grouped_body, ni=ni),
        grid_spec=grid_spec,
        out_shape=jax.ShapeDtypeStruct((R, H), jnp.float32),
        compiler_params=pltpu.CompilerParams(
            dimension_semantics=("arbitrary", "arbitrary"),
            vmem_limit_bytes=100 * 1024 * 1024,
        ),
    )(block_expert, Xs, Wgate, Wup, Wdown, gate_sorted.reshape(R, 1))


def _sc_combine(Ys, pos_a, pos_b, N, H):
    tok_per_w = N // NW
    CH = 32
    nchunk = tok_per_w // CH
    nvec = CH * (H // 16)
    mesh = plsc.VectorSubcoreMesh(core_axis_name="c", subcore_axis_name="s")

    @functools.partial(
        pl.kernel,
        mesh=mesh,
        out_type=jax.ShapeDtypeStruct((N, H), jnp.float32),
        scratch_types=[
            pltpu.VMEM((CH,), jnp.int32),
            pltpu.VMEM((CH,), jnp.int32),
            pltpu.VMEM((CH, H), jnp.float32),
            pltpu.VMEM((CH, H), jnp.float32),
            pltpu.SemaphoreType.DMA,
        ],
    )
    def ck(y_hbm, pa_hbm, pb_hbm, out_hbm, ia_v, ib_v, ba_v, bb_v, sem):
        wid = lax.axis_index("s") * NC + lax.axis_index("c")
        base = wid * tok_per_w

        def chunk(c, carry):
            off = base + c * CH
            pltpu.sync_copy(pa_hbm.at[pl.ds(off, CH)], ia_v)
            pltpu.sync_copy(pb_hbm.at[pl.ds(off, CH)], ib_v)
            cpa = pltpu.async_copy(y_hbm.at[ia_v], ba_v, sem)
            cpb = pltpu.async_copy(y_hbm.at[ib_v], bb_v, sem)
            cpa.wait()
            cpb.wait()

            def add16(k, inner):
                r = k // (H // 16)
                col = (k % (H // 16)) * 16
                ba_v[r, pl.ds(col, 16)] = (
                    ba_v[r, pl.ds(col, 16)] + bb_v[r, pl.ds(col, 16)])
                return inner

            lax.fori_loop(0, nvec, add16, 0)
            pltpu.sync_copy(ba_v, out_hbm.at[pl.ds(off, CH)])
            return carry

        lax.fori_loop(0, nchunk, chunk, 0)

    return ck(Ys, pos_a, pos_b)


def kernel(x, Wg, bg, Wgate, Wup, Wdown):
    N, H = x.shape
    I = Wgate.shape[1]
    R = TOPK * N + E * BT

    logits = _gating_logits(x, Wg, bg)

    l1 = jnp.max(logits, axis=-1)
    i1 = jnp.argmax(logits, axis=-1).astype(jnp.int32)
    eids = jnp.arange(E, dtype=jnp.int32)
    masked = jnp.where(eids[None, :] == i1[:, None], -jnp.inf, logits)
    l2 = jnp.max(masked, axis=-1)
    i2 = jnp.argmax(masked, axis=-1).astype(jnp.int32)

    experts_flat = jnp.concatenate([i1, i2])
    gates_flat = jnp.concatenate([l1, l2])
    tokens_flat = jnp.concatenate(
        [jnp.arange(N, dtype=jnp.int32)] * 2)
    oh = (experts_flat[:, None] == eids[None, :]).astype(jnp.int32)
    csum = jnp.cumsum(oh, axis=0)
    rank = jnp.take_along_axis(csum, experts_flat[:, None], axis=1)[:, 0] - 1
    counts = csum[-1]
    padded = ((counts + BT - 1) // BT) * BT
    starts = jnp.concatenate(
        [jnp.zeros((1,), jnp.int32),
         jnp.cumsum(padded)[:-1].astype(jnp.int32)])
    pos = (starts[experts_flat] + rank).astype(jnp.int32)

    tok_sorted = jnp.zeros((R,), jnp.int32).at[pos].set(tokens_flat)
    gate_sorted = jnp.zeros((R,), jnp.float32).at[pos].set(gates_flat)
    NB = R // BT
    blk_rows = jnp.arange(NB, dtype=jnp.int32) * BT
    block_expert = jnp.clip(
        jnp.sum(blk_rows[:, None] >= starts[None, :], axis=1) - 1,
        0, E - 1).astype(jnp.int32)

    Xs = _sc_gather(x, tok_sorted, R, H)
    Ys = _grouped_mlp(block_expert, Xs, Wgate, Wup, Wdown, gate_sorted,
                      R, H, I)
    final = _sc_combine(Ys, pos[:N], pos[N:], N, H)
    return (final, logits)


# speedup vs baseline: 1.2652x; 1.2652x over previous
import functools

import jax
import jax.numpy as jnp
from jax import lax
from jax.experimental import pallas as pl
from jax.experimental.pallas import tpu as pltpu
from jax.experimental.pallas import tpu_sc as plsc

E = 8
TOPK = 2
BT = 128
I_BLK = 1024

NC = 2
NS = 16
NW = NC * NS


def _logits_body(x_ref, wg_ref, bg_ref, o_ref):
    o_ref[...] = (
        jax.lax.dot_general(
            x_ref[...], wg_ref[...],
            dimension_numbers=(((1,), (1,)), ((), ())),
            preferred_element_type=jnp.float32,
        )
        + bg_ref[...]
    )


def _gating_logits(x, Wg, bg):
    N, H = x.shape
    TB = 256
    return pl.pallas_call(
        _logits_body,
        grid=(N // TB,),
        in_specs=[
            pl.BlockSpec((TB, H), lambda b: (b, 0)),
            pl.BlockSpec((E, H), lambda b: (0, 0)),
            pl.BlockSpec((1, E), lambda b: (0, 0)),
        ],
        out_specs=pl.BlockSpec((TB, E), lambda b: (b, 0)),
        out_shape=jax.ShapeDtypeStruct((N, E), jnp.float32),
    )(x, Wg, bg.reshape(1, E))


def _sc_gather(x, idx, R, H):
    rows_per_w = R // NW
    CH = 48 if rows_per_w % 48 == 0 else 32
    nchunk = rows_per_w // CH
    mesh = plsc.VectorSubcoreMesh(core_axis_name="c", subcore_axis_name="s")

    @functools.partial(
        pl.kernel,
        mesh=mesh,
        out_type=jax.ShapeDtypeStruct((R, H), jnp.float32),
        scratch_types=[
            pltpu.VMEM((CH,), jnp.int32),
            pltpu.VMEM((CH, H), jnp.float32),
            pltpu.SemaphoreType.DMA,
        ],
    )
    def gk(x_hbm, idx_hbm, out_hbm, idx_v, rows_v, sem):
        wid = lax.axis_index("s") * NC + lax.axis_index("c")
        base = wid * rows_per_w

        def chunk(c, carry):
            off = base + c * CH
            pltpu.sync_copy(idx_hbm.at[pl.ds(off, CH)], idx_v)
            pltpu.async_copy(x_hbm.at[idx_v], rows_v, sem).wait()
            pltpu.sync_copy(rows_v, out_hbm.at[pl.ds(off, CH)])
            return carry

        lax.fori_loop(0, nchunk, chunk, 0)

    return gk(x, idx)


def _grouped_body(be_ref, xs_ref, wg_ref, wu_ref, wd_ref, g_ref,
                  yout_ref, yacc_ref, *, ni):
    i = pl.program_id(0)
    b = pl.program_id(1)
    xb = xs_ref[...]
    g1 = jax.lax.dot_general(
        xb, wg_ref[0], (((1,), (1,)), ((), ())),
        preferred_element_type=jnp.float32)
    u = jax.lax.dot_general(
        xb, wu_ref[0], (((1,), (1,)), ((), ())),
        preferred_element_type=jnp.float32)
    h = (g1 * jax.nn.sigmoid(g1)) * u
    part = jax.lax.dot_general(
        h, wd_ref[0], (((1,), (1,)), ((), ())),
        preferred_element_type=jnp.float32)

    sl = pl.ds(b * xs_ref.shape[0], xs_ref.shape[0])

    @pl.when(i == 0)
    def _init():
        yacc_ref[sl, :] = part

    @pl.when((i != 0) & (i != ni - 1))
    def _acc():
        yacc_ref[sl, :] += part

    @pl.when(i == ni - 1)
    def _fin():
        yout_ref[...] = (yacc_ref[sl, :] + part) * g_ref[...]


def _grouped_mlp(block_expert, Xs, Wgate, Wup, Wdown, gate_sorted, R, H, I):
    NB = R // BT
    ni = I // I_BLK
    grid_spec = pltpu.PrefetchScalarGridSpec(
        num_scalar_prefetch=1,
        grid=(ni, NB),
        in_specs=[
            pl.BlockSpec((BT, H), lambda i, b, be: (b, 0)),
            pl.BlockSpec((1, I_BLK, H), lambda i, b, be: (be[b], i, 0)),
            pl.BlockSpec((1, I_BLK, H), lambda i, b, be: (be[b], i, 0)),
            pl.BlockSpec((1, H, I_BLK), lambda i, b, be: (be[b], 0, i)),
            pl.BlockSpec((BT, 1), lambda i, b, be: (b, 0)),
        ],
        out_specs=pl.BlockSpec((BT, H), lambda i, b, be: (b, 0)),
        scratch_shapes=[pltpu.VMEM((R, H), jnp.float32)],
    )
    return pl.pallas_call(
        functools.partial(_grouped_body, ni=ni),
        grid_spec=grid_spec,
        out_shape=jax.ShapeDtypeStruct((R, H), jnp.float32),
        compiler_params=pltpu.CompilerParams(
            dimension_semantics=("arbitrary", "arbitrary"),
            vmem_limit_bytes=100 * 1024 * 1024,
        ),
    )(block_expert, Xs, Wgate, Wup, Wdown, gate_sorted.reshape(R, 1))


def _sc_combine(Ys, pos_a, pos_b, N, H):
    tok_per_w = N // NW
    CH = 32
    nchunk = tok_per_w // CH
    nvec = CH * (H // 16)
    mesh = plsc.VectorSubcoreMesh(core_axis_name="c", subcore_axis_name="s")

    @functools.partial(
        pl.kernel,
        mesh=mesh,
        out_type=jax.ShapeDtypeStruct((N, H), jnp.float32),
        scratch_types=[
            pltpu.VMEM((CH,), jnp.int32),
            pltpu.VMEM((CH,), jnp.int32),
            pltpu.VMEM((CH, H), jnp.float32),
            pltpu.VMEM((CH, H), jnp.float32),
            pltpu.SemaphoreType.DMA,
        ],
    )
    def ck(y_hbm, pa_hbm, pb_hbm, out_hbm, ia_v, ib_v, ba_v, bb_v, sem):
        wid = lax.axis_index("s") * NC + lax.axis_index("c")
        base = wid * tok_per_w

        def chunk(c, carry):
            off = base + c * CH
            pltpu.sync_copy(pa_hbm.at[pl.ds(off, CH)], ia_v)
            pltpu.sync_copy(pb_hbm.at[pl.ds(off, CH)], ib_v)
            cpa = pltpu.async_copy(y_hbm.at[ia_v], ba_v, sem)
            cpb = pltpu.async_copy(y_hbm.at[ib_v], bb_v, sem)
            cpa.wait()
            cpb.wait()

            def add16(k, inner):
                r = k // (H // 16)
                col = (k % (H // 16)) * 16
                ba_v[r, pl.ds(col, 16)] = (
                    ba_v[r, pl.ds(col, 16)] + bb_v[r, pl.ds(col, 16)])
                return inner

            lax.fori_loop(0, nvec, add16, 0)
            pltpu.sync_copy(ba_v, out_hbm.at[pl.ds(off, CH)])
            return carry

        lax.fori_loop(0, nchunk, chunk, 0)

    return ck(Ys, pos_a, pos_b)


def kernel(x, Wg, bg, Wgate, Wup, Wdown):
    N, H = x.shape
    I = Wgate.shape[1]
    R = TOPK * N + E * BT

    logits = _gating_logits(x, Wg, bg)

    l1 = jnp.max(logits, axis=-1)
    i1 = jnp.argmax(logits, axis=-1).astype(jnp.int32)
    eids = jnp.arange(E, dtype=jnp.int32)
    masked = jnp.where(eids[None, :] == i1[:, None], -jnp.inf, logits)
    l2 = jnp.max(masked, axis=-1)
    i2 = jnp.argmax(masked, axis=-1).astype(jnp.int32)

    experts_flat = jnp.concatenate([i1, i2])
    gates_flat = jnp.concatenate([l1, l2])
    tokens_flat = jnp.concatenate(
        [jnp.arange(N, dtype=jnp.int32)] * 2)
    oh = (experts_flat[:, None] == eids[None, :]).astype(jnp.int32)
    csum = jnp.cumsum(oh, axis=0)
    rank = jnp.take_along_axis(csum, experts_flat[:, None], axis=1)[:, 0] - 1
    counts = csum[-1]
    padded = ((counts + BT - 1) // BT) * BT
    starts = jnp.concatenate(
        [jnp.zeros((1,), jnp.int32),
         jnp.cumsum(padded)[:-1].astype(jnp.int32)])
    pos = (starts[experts_flat] + rank).astype(jnp.int32)

    tok_sorted = jnp.zeros((R,), jnp.int32).at[pos].set(tokens_flat)
    gate_sorted = jnp.zeros((R,), jnp.float32).at[pos].set(gates_flat)
    NB = R // BT
    blk_rows = jnp.arange(NB, dtype=jnp.int32) * BT
    block_expert = jnp.clip(
        jnp.sum(blk_rows[:, None] >= starts[None, :], axis=1) - 1,
        0, E - 1).astype(jnp.int32)

    Xs = _sc_gather(x, tok_sorted, R, H)
    Ys = _grouped_mlp(block_expert, Xs, Wgate, Wup, Wdown, gate_sorted,
                      R, H, I)
    final = _sc_combine(Ys, pos[:N], pos[N:], N, H)
    return (final, logits)
